# baseline (speedup 1.0000x reference)
import functools

import jax
import jax.numpy as jnp
from jax.experimental import pallas as pl
from jax.experimental.pallas import tpu as pltpu

_K = 256
_ROWS_PER_BLOCK = 32
_MAX_INTERP_STEPS = 40


def _key_to_f32(u):
    s = jnp.where(u >= jnp.uint32(0x80000000), u ^ jnp.uint32(0x80000000), ~u)
    return jax.lax.bitcast_convert_type(s, jnp.float32)


def _f32_to_key(x):
    s = jax.lax.bitcast_convert_type(x, jnp.uint32)
    return jnp.where(s >= jnp.uint32(0x80000000), ~s, s | jnp.uint32(0x80000000))


def _topk_mask_kernel(x_ref, out_ref, k):
    cols = x_ref.shape[1]
    n_sl = cols // 256
    kf = jnp.float32(k)

    accs = [x_ref[:, 256 * i:256 * (i + 1)] for i in range(4)]
    for i in range(4, n_sl):
        accs[i % 4] = jnp.maximum(accs[i % 4], x_ref[:, 256 * i:256 * (i + 1)])
    g = jnp.maximum(jnp.maximum(accs[0], accs[1]),
                    jnp.maximum(accs[2], accs[3]))
    f_lo0 = jnp.min(g, axis=1, keepdims=True)
    lo0 = _f32_to_key(f_lo0)
    hi0 = _f32_to_key(jnp.max(g, axis=1, keepdims=True)) + jnp.uint32(1)

    def count(t):
        return jnp.sum((x_ref[...] >= t).astype(jnp.float32), axis=1,
                       keepdims=True)

    cnt_lo0 = count(f_lo0)
    cnt_hi0 = jnp.zeros_like(cnt_lo0)

    def unconverged(lo, hi, cnt_lo):
        return (cnt_lo != kf) & ((hi - lo) > jnp.uint32(1))

    def probe(mid, carry):
        lo, hi, cnt_lo, cnt_hi = carry
        cnt = count(_key_to_f32(mid))
        ge = cnt >= kf
        return (jnp.where(ge, mid, lo), jnp.where(ge, hi, mid),
                jnp.where(ge, cnt, cnt_lo), jnp.where(ge, cnt_hi, cnt))

    def cond_i(state):
        it, carry = state
        lo, hi, cnt_lo, _ = carry
        return (it < _MAX_INTERP_STEPS) & jnp.any(unconverged(lo, hi, cnt_lo))

    def body_i(state):
        it, carry = state
        lo, hi, cnt_lo, cnt_hi = carry
        f_lo = _key_to_f32(lo)
        f_hi = _key_to_f32(hi)
        frac = (cnt_lo - kf) / (cnt_lo - cnt_hi)
        kg = _f32_to_key(f_lo + frac * (f_hi - f_lo))
        bias = jnp.uint32(0x80000000)

        def _s(u):
            return jax.lax.bitcast_convert_type(u ^ bias, jnp.int32)

        mids = jnp.minimum(jnp.maximum(_s(kg), _s(lo + jnp.uint32(1))),
                           _s(hi - jnp.uint32(1)))
        mid = jax.lax.bitcast_convert_type(mids, jnp.uint32) ^ bias
        return it + 1, probe(mid, carry)

    _, carry = jax.lax.while_loop(
        cond_i, body_i, (jnp.int32(0), (lo0, hi0, cnt_lo0, cnt_hi0)))

    def cond_b(carry):
        lo, hi, cnt_lo, _ = carry
        return jnp.any(unconverged(lo, hi, cnt_lo))

    def body_b(carry):
        lo, hi, _, _ = carry
        mid = lo + ((hi - lo) >> jnp.uint32(1))
        return probe(mid, carry)

    lo, _, _, _ = jax.lax.while_loop(cond_b, body_b, carry)
    t = _key_to_f32(lo)
    x = x_ref[...]
    out_ref[...] = jnp.where((x >= t) & (x > 0.0), x, 0.0)


def kernel(x):
    n_rows, n_cols = x.shape
    r = _ROWS_PER_BLOCK
    grid = (n_rows // r,)
    return pl.pallas_call(
        functools.partial(_topk_mask_kernel, k=_K),
        grid=grid,
        in_specs=[pl.BlockSpec((r, n_cols), lambda i: (i, 0))],
        out_specs=pl.BlockSpec((r, n_cols), lambda i: (i, 0)),
        out_shape=jax.ShapeDtypeStruct(x.shape, x.dtype),
    )(x)


# speedup vs baseline: 1.8922x; 1.8922x over previous
import functools

import jax
import jax.numpy as jnp
from jax.experimental import pallas as pl
from jax.experimental.pallas import tpu as pltpu

_K = 256
_ROWS_PER_BLOCK = 32
_MAX_INTERP_STEPS = 40


def _key_to_f32(u):
    s = jnp.where(u >= jnp.uint32(0x80000000), u ^ jnp.uint32(0x80000000), ~u)
    return jax.lax.bitcast_convert_type(s, jnp.float32)


def _f32_to_key(x):
    s = jax.lax.bitcast_convert_type(x, jnp.uint32)
    return jnp.where(s >= jnp.uint32(0x80000000), ~s, s | jnp.uint32(0x80000000))


def _topk_mask_kernel(x_ref, out_ref, k):
    cols = x_ref.shape[1]
    n_sl = cols // 256
    kf = jnp.float32(k)

    accs = [x_ref[:, 256 * i:256 * (i + 1)] for i in range(4)]
    for i in range(4, n_sl):
        accs[i % 4] = jnp.maximum(accs[i % 4], x_ref[:, 256 * i:256 * (i + 1)])
    g = jnp.maximum(jnp.maximum(accs[0], accs[1]),
                    jnp.maximum(accs[2], accs[3]))
    f_lo0 = jnp.min(g, axis=1, keepdims=True)
    lo0 = _f32_to_key(f_lo0)
    hi0 = _f32_to_key(jnp.max(g, axis=1, keepdims=True)) + jnp.uint32(1)

    def count(t):
        return jnp.sum((x_ref[...] >= t).astype(jnp.float32), axis=1,
                       keepdims=True)

    cnt_lo0 = count(f_lo0)
    cnt_hi0 = jnp.zeros_like(cnt_lo0)

    def unconverged(lo, hi, cnt_lo):
        return (cnt_lo != kf) & ((hi - lo) > jnp.uint32(1))

    def probe(mid, carry):
        lo, hi, cnt_lo, cnt_hi = carry
        cnt = count(_key_to_f32(mid))
        ge = cnt >= kf
        return (jnp.where(ge, mid, lo), jnp.where(ge, hi, mid),
                jnp.where(ge, cnt, cnt_lo), jnp.where(ge, cnt_hi, cnt))

    def cond_i(state):
        it, carry = state
        lo, hi, cnt_lo, _ = carry
        return (it < _MAX_INTERP_STEPS) & jnp.any(unconverged(lo, hi, cnt_lo))

    def body_i(state):
        it, carry = state
        lo, hi, cnt_lo, cnt_hi = carry
        f_lo = _key_to_f32(lo)
        f_hi = _key_to_f32(hi)
        llo = jnp.log(cnt_lo)
        lhi = jnp.log(jnp.maximum(cnt_hi, jnp.float32(0.4)))
        frac = (llo - jnp.log(kf)) / jnp.maximum(llo - lhi, jnp.float32(1e-6))
        frac = jnp.clip(frac, 0.0, 1.0)
        kg = _f32_to_key(f_lo + frac * (f_hi - f_lo))
        bias = jnp.uint32(0x80000000)

        def _s(u):
            return jax.lax.bitcast_convert_type(u ^ bias, jnp.int32)

        mids = jnp.minimum(jnp.maximum(_s(kg), _s(lo + jnp.uint32(1))),
                           _s(hi - jnp.uint32(1)))
        mid = jax.lax.bitcast_convert_type(mids, jnp.uint32) ^ bias
        return it + 1, probe(mid, carry)

    _, carry = jax.lax.while_loop(
        cond_i, body_i, (jnp.int32(0), (lo0, hi0, cnt_lo0, cnt_hi0)))

    def cond_b(carry):
        lo, hi, cnt_lo, _ = carry
        return jnp.any(unconverged(lo, hi, cnt_lo))

    def body_b(carry):
        lo, hi, _, _ = carry
        mid = lo + ((hi - lo) >> jnp.uint32(1))
        return probe(mid, carry)

    lo, _, _, _ = jax.lax.while_loop(cond_b, body_b, carry)
    t = _key_to_f32(lo)
    x = x_ref[...]
    out_ref[...] = jnp.where((x >= t) & (x > 0.0), x, 0.0)


def kernel(x):
    n_rows, n_cols = x.shape
    r = _ROWS_PER_BLOCK
    grid = (n_rows // r,)
    return pl.pallas_call(
        functools.partial(_topk_mask_kernel, k=_K),
        grid=grid,
        in_specs=[pl.BlockSpec((r, n_cols), lambda i: (i, 0))],
        out_specs=pl.BlockSpec((r, n_cols), lambda i: (i, 0)),
        out_shape=jax.ShapeDtypeStruct(x.shape, x.dtype),
    )(x)


# speedup vs baseline: 2.1742x; 1.1490x over previous
import functools

import jax
import jax.numpy as jnp
from jax.experimental import pallas as pl
from jax.experimental.pallas import tpu as pltpu

_K = 256
_ROWS_PER_BLOCK = 64
_MAX_INTERP_STEPS = 40


def _key_to_f32(u):
    s = jnp.where(u >= jnp.uint32(0x80000000), u ^ jnp.uint32(0x80000000), ~u)
    return jax.lax.bitcast_convert_type(s, jnp.float32)


def _f32_to_key(x):
    s = jax.lax.bitcast_convert_type(x, jnp.uint32)
    return jnp.where(s >= jnp.uint32(0x80000000), ~s, s | jnp.uint32(0x80000000))


def _topk_mask_kernel(x_ref, out_ref, k):
    cols = x_ref.shape[1]
    n_sl = cols // 256
    kf = jnp.float32(k)

    accs = [x_ref[:, 256 * i:256 * (i + 1)] for i in range(4)]
    for i in range(4, n_sl):
        accs[i % 4] = jnp.maximum(accs[i % 4], x_ref[:, 256 * i:256 * (i + 1)])
    g = jnp.maximum(jnp.maximum(accs[0], accs[1]),
                    jnp.maximum(accs[2], accs[3]))
    f_lo0 = jnp.min(g, axis=1, keepdims=True)
    lo0 = _f32_to_key(f_lo0)
    hi0 = _f32_to_key(jnp.max(g, axis=1, keepdims=True)) + jnp.uint32(1)

    def count(t):
        return jnp.sum((x_ref[...] >= t).astype(jnp.float32), axis=1,
                       keepdims=True)

    cnt_lo0 = count(f_lo0)
    cnt_hi0 = jnp.zeros_like(cnt_lo0)

    def unconverged(lo, hi, cnt_lo):
        return (cnt_lo != kf) & ((hi - lo) > jnp.uint32(1))

    def probe(mid, carry):
        lo, hi, cnt_lo, cnt_hi = carry
        cnt = count(_key_to_f32(mid))
        ge = cnt >= kf
        return (jnp.where(ge, mid, lo), jnp.where(ge, hi, mid),
                jnp.where(ge, cnt, cnt_lo), jnp.where(ge, cnt_hi, cnt))

    def cond_i(state):
        it, carry = state
        lo, hi, cnt_lo, _ = carry
        return (it < _MAX_INTERP_STEPS) & jnp.any(unconverged(lo, hi, cnt_lo))

    def body_i(state):
        it, carry = state
        lo, hi, cnt_lo, cnt_hi = carry
        f_lo = _key_to_f32(lo)
        f_hi = _key_to_f32(hi)
        llo = jnp.log(cnt_lo)
        lhi = jnp.log(jnp.maximum(cnt_hi, jnp.float32(0.4)))
        frac = (llo - jnp.log(kf)) / jnp.maximum(llo - lhi, jnp.float32(1e-6))
        frac = jnp.clip(frac, 0.0, 1.0)
        kg = _f32_to_key(f_lo + frac * (f_hi - f_lo))
        bias = jnp.uint32(0x80000000)

        def _s(u):
            return jax.lax.bitcast_convert_type(u ^ bias, jnp.int32)

        mids = jnp.minimum(jnp.maximum(_s(kg), _s(lo + jnp.uint32(1))),
                           _s(hi - jnp.uint32(1)))
        mid = jax.lax.bitcast_convert_type(mids, jnp.uint32) ^ bias
        return it + 1, probe(mid, carry)

    _, carry = jax.lax.while_loop(
        cond_i, body_i, (jnp.int32(0), (lo0, hi0, cnt_lo0, cnt_hi0)))

    def cond_b(carry):
        lo, hi, cnt_lo, _ = carry
        return jnp.any(unconverged(lo, hi, cnt_lo))

    def body_b(carry):
        lo, hi, _, _ = carry
        mid = lo + ((hi - lo) >> jnp.uint32(1))
        return probe(mid, carry)

    lo, _, _, _ = jax.lax.while_loop(cond_b, body_b, carry)
    t = _key_to_f32(lo)
    x = x_ref[...]
    out_ref[...] = jnp.where((x >= t) & (x > 0.0), x, 0.0)


def kernel(x):
    n_rows, n_cols = x.shape
    r = _ROWS_PER_BLOCK
    grid = (n_rows // r,)
    return pl.pallas_call(
        functools.partial(_topk_mask_kernel, k=_K),
        grid=grid,
        in_specs=[pl.BlockSpec((r, n_cols), lambda i: (i, 0))],
        out_specs=pl.BlockSpec((r, n_cols), lambda i: (i, 0)),
        out_shape=jax.ShapeDtypeStruct(x.shape, x.dtype),
    )(x)
